# Initial kernel scaffold; baseline (speedup 1.0000x reference)
import jax
import jax.numpy as jnp
from jax.experimental import pallas as pl


def kernel(atoms, bonds, edges, W, b):
    raise NotImplementedError("write your pallas kernel here")


# speedup vs baseline: 3.0646x; 3.0646x over previous
import functools

import jax
import jax.numpy as jnp
from jax import lax
from jax.experimental import pallas as pl
from jax.experimental.pallas import tpu as pltpu
from jax.experimental.pallas import tpu_sc as plsc

B, A, D, F, FB, C = 128, 512, 8, 64, 16, 64

NUM_CORES = 2
NUM_SUBCORES = 16
NUM_WORKERS = NUM_CORES * NUM_SUBCORES
SAMPLES_PER_WORKER = B // NUM_WORKERS
CHUNK_ATOMS = 16
CHUNK_IDX = CHUNK_ATOMS * D
CHUNKS_PER_SAMPLE = A // CHUNK_ATOMS
ZERO_ROW = B * A


def _gather_body(table_hbm, edges_hbm, out_hbm, edges_v, idx_v, rows_v, acc_v, sem):
    wid = lax.axis_index("s") * NUM_CORES + lax.axis_index("c")

    def per_sample(s, _):
        b = wid * SAMPLES_PER_WORKER + s
        pltpu.sync_copy(edges_hbm.at[b], edges_v)
        base = b * A

        def per_chunk(c, _):
            def per_vreg(k, _):
                e = edges_v[pl.ds(c * CHUNK_IDX + k * 16, 16)]
                idx_v[pl.ds(k * 16, 16)] = jnp.where(e < 0, ZERO_ROW, e + base)
                return 0

            lax.fori_loop(0, CHUNK_IDX // 16, per_vreg, 0)
            pltpu.async_copy(table_hbm.at[idx_v], rows_v, sem).wait()

            def per_atom(a, _):
                r0 = a * D
                for j in range(F // 16):
                    sl = pl.ds(j * 16, 16)
                    v = rows_v[r0, sl]
                    for d in range(1, D):
                        v = v + rows_v[r0 + d, sl]
                    acc_v[c * CHUNK_ATOMS + a, sl] = v
                return 0

            lax.fori_loop(0, CHUNK_ATOMS, per_atom, 0)
            return 0

        lax.fori_loop(0, CHUNKS_PER_SAMPLE, per_chunk, 0)
        pltpu.sync_copy(acc_v, out_hbm.at[pl.ds(base, A)])
        return 0

    lax.fori_loop(0, SAMPLES_PER_WORKER, per_sample, 0)


@functools.cache
def _sc_gather():
    return functools.partial(
        pl.kernel,
        mesh=plsc.VectorSubcoreMesh(core_axis_name="c", subcore_axis_name="s"),
        compiler_params=pltpu.CompilerParams(use_tc_tiling_on_sc=False),
        out_type=jax.ShapeDtypeStruct((B * A, F), jnp.float32),
        scratch_types=[
            pltpu.VMEM((A * D,), jnp.int32),
            pltpu.VMEM((CHUNK_IDX,), jnp.int32),
            pltpu.VMEM((CHUNK_IDX, F), jnp.float32),
            pltpu.VMEM((A, F), jnp.float32),
            pltpu.SemaphoreType.DMA,
        ],
    )(_gather_body)


ROWS_BLK = 512


def _dense_body(neigh_ref, atoms_ref, bonds_ref, edges_ref, wa_ref, wb_ref,
                bias_ref, out_ref):
    na = neigh_ref[...] + atoms_ref[...]
    z = jnp.dot(na, wa_ref[...], preferred_element_type=jnp.float32)
    z = z + jnp.dot(bonds_ref[...], wb_ref[...], preferred_element_type=jnp.float32)
    z = z + bias_ref[...]
    deg = jnp.sum((edges_ref[...] >= 0).astype(jnp.int32), axis=1, keepdims=True)
    acc = jnp.zeros((ROWS_BLK, C), jnp.float32)
    for d in range(D):
        sl = z[:, d * C:(d + 1) * C]
        acc = acc + jnp.where(deg == d, jnp.maximum(sl, 0.0), 0.0)
    out_ref[...] = acc


def kernel(atoms, bonds, edges, W, b):
    atoms_f = atoms.reshape(B * A, F)
    table = jnp.concatenate([atoms_f, jnp.zeros((1, F), jnp.float32)], axis=0)
    edges_sc = edges.reshape(B, A * D)

    neigh = _sc_gather()(table, edges_sc)

    bonds_f = bonds.reshape(B * A, D * FB)
    edges_f = edges.reshape(B * A, D)
    wa = jnp.transpose(W[:, :F, :], (1, 0, 2)).reshape(F, D * C)
    wb = jnp.tile(jnp.transpose(W[:, F:, :], (1, 0, 2)).reshape(FB, D * C), (D, 1))
    bias = b.reshape(1, D * C)

    grid = (B * A // ROWS_BLK,)
    out = pl.pallas_call(
        _dense_body,
        grid=grid,
        in_specs=[
            pl.BlockSpec((ROWS_BLK, F), lambda i: (i, 0)),
            pl.BlockSpec((ROWS_BLK, F), lambda i: (i, 0)),
            pl.BlockSpec((ROWS_BLK, D * FB), lambda i: (i, 0)),
            pl.BlockSpec((ROWS_BLK, D), lambda i: (i, 0)),
            pl.BlockSpec((F, D * C), lambda i: (0, 0)),
            pl.BlockSpec((D * FB, D * C), lambda i: (0, 0)),
            pl.BlockSpec((1, D * C), lambda i: (0, 0)),
        ],
        out_specs=pl.BlockSpec((ROWS_BLK, C), lambda i: (i, 0)),
        out_shape=jax.ShapeDtypeStruct((B * A, C), jnp.float32),
    )(neigh, atoms_f, bonds_f, edges_f, wa, wb, bias)

    return out.reshape(B, A, C)


# speedup vs baseline: 17.1760x; 5.6047x over previous
import functools

import jax
import jax.numpy as jnp
from jax import lax
from jax.experimental import pallas as pl
from jax.experimental.pallas import tpu as pltpu
from jax.experimental.pallas import tpu_sc as plsc

B, A, D, F, FB, C = 128, 512, 8, 64, 16, 64

NUM_CORES = 2
NUM_SUBCORES = 16
NUM_WORKERS = NUM_CORES * NUM_SUBCORES
SAMPLES_PER_WORKER = B // NUM_WORKERS
CHUNK_ATOMS = 16
CHUNK_IDX = CHUNK_ATOMS * D
CHUNKS_PER_SAMPLE = A // CHUNK_ATOMS
ZERO_ROW = B * A


def _gather_body(table_hbm, edges_hbm, out_hbm,
                 tab0, tab1, edg0, edg1, out_v,
                 sem_t0, sem_e0, sem_t1, sem_e1):
    wid = lax.axis_index("s") * NUM_CORES + lax.axis_index("c")
    ai = lax.iota(jnp.int32, 16)

    tabs = (tab0, tab1)
    edgs = (edg0, edg1)
    sems = ((sem_t0, sem_e0), (sem_t1, sem_e1))

    for t in tabs:
        for j in range(F // 16):
            t[pl.ds(A * F + j * 16, 16)] = jnp.zeros((16,), jnp.float32)

    def stage(s, slot):
        b = wid * SAMPLES_PER_WORKER + s
        ct = pltpu.async_copy(
            table_hbm.at[pl.ds(b * A * F, A * F)],
            tabs[slot].at[pl.ds(0, A * F)], sems[slot][0])
        ce = pltpu.async_copy(edges_hbm.at[b], edgs[slot], sems[slot][1])
        return (ct, ce)

    cps = stage(0, 0)
    for s in range(SAMPLES_PER_WORKER):
        slot = s & 1
        tab, edg = tabs[slot], edgs[slot]
        for c in cps:
            c.wait()
        if s + 1 < SAMPLES_PER_WORKER:
            cps = stage(s + 1, slot ^ 1)

        def per_chunk(ch, _):
            e_base = ch * CHUNK_IDX + ai * D
            addrs = []
            for d in range(D):
                e_d = plsc.load_gather(edg, [e_base + d])
                addrs.append(jnp.where(e_d < 0, A, e_d) * F)
            out_base = ch * CHUNK_ATOMS * F + ai * F

            def per_f(f, _):
                v = plsc.load_gather(tab, [addrs[0] + f])
                for d in range(1, D):
                    v = v + plsc.load_gather(tab, [addrs[d] + f])
                plsc.store_scatter(out_v, [out_base + f], v)
                return 0

            lax.fori_loop(0, F, per_f, 0)
            return 0

        lax.fori_loop(0, CHUNKS_PER_SAMPLE, per_chunk, 0)
        b = wid * SAMPLES_PER_WORKER + s
        pltpu.sync_copy(out_v, out_hbm.at[pl.ds(b * A * F, A * F)])


@functools.cache
def _sc_gather():
    return functools.partial(
        pl.kernel,
        mesh=plsc.VectorSubcoreMesh(core_axis_name="c", subcore_axis_name="s"),
        compiler_params=pltpu.CompilerParams(
            use_tc_tiling_on_sc=False, needs_layout_passes=False),
        out_type=jax.ShapeDtypeStruct((B * A * F,), jnp.float32),
        scratch_types=[
            pltpu.VMEM(((A + 1) * F,), jnp.float32),
            pltpu.VMEM(((A + 1) * F,), jnp.float32),
            pltpu.VMEM((A * D,), jnp.int32),
            pltpu.VMEM((A * D,), jnp.int32),
            pltpu.VMEM((A * F,), jnp.float32),
            pltpu.SemaphoreType.DMA,
            pltpu.SemaphoreType.DMA,
            pltpu.SemaphoreType.DMA,
            pltpu.SemaphoreType.DMA,
        ],
    )(_gather_body)


ROWS_BLK = 512


def _dense_body(neigh_ref, atoms_ref, bonds_ref, edges_ref, wa_ref, wb_ref,
                bias_ref, out_ref):
    na = neigh_ref[...] + atoms_ref[...]
    z = jnp.dot(na, wa_ref[...], preferred_element_type=jnp.float32)
    z = z + jnp.dot(bonds_ref[...], wb_ref[...], preferred_element_type=jnp.float32)
    z = z + bias_ref[...]
    deg = jnp.sum((edges_ref[...] >= 0).astype(jnp.int32), axis=1, keepdims=True)
    acc = jnp.zeros((ROWS_BLK, C), jnp.float32)
    for d in range(D):
        sl = z[:, d * C:(d + 1) * C]
        acc = acc + jnp.where(deg == d, jnp.maximum(sl, 0.0), 0.0)
    out_ref[...] = acc


def kernel(atoms, bonds, edges, W, b):
    atoms_f = atoms.reshape(B * A, F)
    edges_sc = edges.reshape(B, A * D)

    neigh = _sc_gather()(atoms.reshape(B * A * F), edges_sc).reshape(B * A, F)

    bonds_f = bonds.reshape(B * A, D * FB)
    edges_f = edges.reshape(B * A, D)
    wa = jnp.transpose(W[:, :F, :], (1, 0, 2)).reshape(F, D * C)
    wb = jnp.tile(jnp.transpose(W[:, F:, :], (1, 0, 2)).reshape(FB, D * C), (D, 1))
    bias = b.reshape(1, D * C)

    grid = (B * A // ROWS_BLK,)
    out = pl.pallas_call(
        _dense_body,
        grid=grid,
        in_specs=[
            pl.BlockSpec((ROWS_BLK, F), lambda i: (i, 0)),
            pl.BlockSpec((ROWS_BLK, F), lambda i: (i, 0)),
            pl.BlockSpec((ROWS_BLK, D * FB), lambda i: (i, 0)),
            pl.BlockSpec((ROWS_BLK, D), lambda i: (i, 0)),
            pl.BlockSpec((F, D * C), lambda i: (0, 0)),
            pl.BlockSpec((D * FB, D * C), lambda i: (0, 0)),
            pl.BlockSpec((1, D * C), lambda i: (0, 0)),
        ],
        out_specs=pl.BlockSpec((ROWS_BLK, C), lambda i: (i, 0)),
        out_shape=jax.ShapeDtypeStruct((B * A, C), jnp.float32),
    )(neigh, atoms_f, bonds_f, edges_f, wa, wb, bias)

    return out.reshape(B, A, C)


# speedup vs baseline: 36.4526x; 2.1223x over previous
import functools

import jax
import jax.numpy as jnp
from jax import lax
from jax.experimental import pallas as pl
from jax.experimental.pallas import tpu as pltpu
from jax.experimental.pallas import tpu_sc as plsc

B, A, D, F, FB, C = 128, 512, 8, 64, 16, 64

NUM_CORES = 2
NUM_SUBCORES = 16
NUM_WORKERS = NUM_CORES * NUM_SUBCORES
SAMPLES_PER_WORKER = B // NUM_WORKERS
CHUNK_ATOMS = 16
CHUNK_IDX = CHUNK_ATOMS * D
CHUNKS_PER_SAMPLE = A // CHUNK_ATOMS
AP = A + 16


def _gather_body(table_hbm, edges_hbm, out_hbm,
                 tab0, tab1, edg0, edg1, out_v,
                 sem_t0, sem_e0, sem_t1, sem_e1):
    wid = lax.axis_index("s") * NUM_CORES + lax.axis_index("c")
    ai = lax.iota(jnp.int32, 16)

    tabs = (tab0, tab1)
    edgs = (edg0, edg1)
    sems = ((sem_t0, sem_e0), (sem_t1, sem_e1))

    def stage(s, slot):
        b = wid * SAMPLES_PER_WORKER + s
        ct = pltpu.async_copy(table_hbm.at[b], tabs[slot], sems[slot][0])
        ce = pltpu.async_copy(edges_hbm.at[b], edgs[slot], sems[slot][1])
        return (ct, ce)

    cps = stage(0, 0)
    for s in range(SAMPLES_PER_WORKER):
        slot = s & 1
        tab, edg = tabs[slot], edgs[slot]
        for c in cps:
            c.wait()
        if s + 1 < SAMPLES_PER_WORKER:
            cps = stage(s + 1, slot ^ 1)

        def per_chunk(ch, _):
            e_base = ch * CHUNK_IDX + ai * D
            cols = []
            for d in range(D):
                e_d = plsc.load_gather(edg, [e_base + d])
                cols.append(jnp.where(e_d < 0, A + ai, e_d))
            a0 = ch * CHUNK_ATOMS

            def per_f(f, _):
                base = f * AP
                v = tab[pl.ds(base + a0, 16)]
                for d in range(D):
                    v = v + plsc.load_gather(tab, [cols[d] + base])
                out_v[pl.ds(f * A + a0, 16)] = v
                return 0

            lax.fori_loop(0, F, per_f, 0)
            return 0

        lax.fori_loop(0, CHUNKS_PER_SAMPLE, per_chunk, 0)
        b = wid * SAMPLES_PER_WORKER + s
        pltpu.sync_copy(out_v, out_hbm.at[b])


@functools.cache
def _sc_gather():
    return functools.partial(
        pl.kernel,
        mesh=plsc.VectorSubcoreMesh(core_axis_name="c", subcore_axis_name="s"),
        compiler_params=pltpu.CompilerParams(
            use_tc_tiling_on_sc=False, needs_layout_passes=False),
        out_type=jax.ShapeDtypeStruct((B, F * A), jnp.float32),
        scratch_types=[
            pltpu.VMEM((F * AP,), jnp.float32),
            pltpu.VMEM((F * AP,), jnp.float32),
            pltpu.VMEM((A * D,), jnp.int32),
            pltpu.VMEM((A * D,), jnp.int32),
            pltpu.VMEM((F * A,), jnp.float32),
            pltpu.SemaphoreType.DMA,
            pltpu.SemaphoreType.DMA,
            pltpu.SemaphoreType.DMA,
            pltpu.SemaphoreType.DMA,
        ],
    )(_gather_body)


ROWS_BLK = A


def _dense_body(neigh_ref, bonds_ref, edges_ref, wa_ref, wb_ref,
                bias_ref, out_ref):
    z = lax.dot_general(neigh_ref[...], wa_ref[...],
                        (((0,), (0,)), ((), ())),
                        preferred_element_type=jnp.float32)
    z = z + jnp.dot(bonds_ref[...], wb_ref[...],
                    preferred_element_type=jnp.float32)
    z = z + bias_ref[...]
    deg = jnp.sum((edges_ref[...] >= 0).astype(jnp.int32), axis=1, keepdims=True)
    acc = jnp.zeros((ROWS_BLK, C), jnp.float32)
    for d in range(D):
        sl = z[:, d * C:(d + 1) * C]
        acc = acc + jnp.where(deg == d, jnp.maximum(sl, 0.0), 0.0)
    out_ref[...] = acc


def kernel(atoms, bonds, edges, W, b):
    atoms_tp = jnp.pad(atoms.transpose(0, 2, 1), ((0, 0), (0, 0), (0, AP - A)))
    edges_sc = edges.reshape(B, A * D)

    neigh_t = _sc_gather()(atoms_tp.reshape(B, F * AP), edges_sc)
    neigh_t = neigh_t.reshape(B * F, A)

    bonds_f = bonds.reshape(B * A, D * FB)
    edges_f = edges.reshape(B * A, D)
    wa = jnp.transpose(W[:, :F, :], (1, 0, 2)).reshape(F, D * C)
    wb = jnp.tile(jnp.transpose(W[:, F:, :], (1, 0, 2)).reshape(FB, D * C), (D, 1))
    bias = b.reshape(1, D * C)

    grid = (B * A // ROWS_BLK,)
    out = pl.pallas_call(
        _dense_body,
        grid=grid,
        in_specs=[
            pl.BlockSpec((F, ROWS_BLK), lambda i: (i, 0)),
            pl.BlockSpec((ROWS_BLK, D * FB), lambda i: (i, 0)),
            pl.BlockSpec((ROWS_BLK, D), lambda i: (i, 0)),
            pl.BlockSpec((F, D * C), lambda i: (0, 0)),
            pl.BlockSpec((D * FB, D * C), lambda i: (0, 0)),
            pl.BlockSpec((1, D * C), lambda i: (0, 0)),
        ],
        out_specs=pl.BlockSpec((ROWS_BLK, C), lambda i: (i, 0)),
        out_shape=jax.ShapeDtypeStruct((B * A, C), jnp.float32),
    )(neigh_t, bonds_f, edges_f, wa, wb, bias)

    return out.reshape(B, A, C)


# speedup vs baseline: 36.8058x; 1.0097x over previous
import functools

import jax
import jax.numpy as jnp
from jax import lax
from jax.experimental import pallas as pl
from jax.experimental.pallas import tpu as pltpu
from jax.experimental.pallas import tpu_sc as plsc

B, A, D, F, FB, C = 128, 512, 8, 64, 16, 64

NUM_CORES = 2
NUM_SUBCORES = 16
NUM_WORKERS = NUM_CORES * NUM_SUBCORES
SAMPLES_PER_WORKER = B // NUM_WORKERS
CHUNK_ATOMS = 16
CHUNK_IDX = CHUNK_ATOMS * D
CHUNKS_PER_SAMPLE = A // CHUNK_ATOMS
AP = A + 16


def _gather_body(spw, table_hbm, edges_hbm, out_hbm,
                 tab0, tab1, edg0, edg1, out_v,
                 sem_t0, sem_e0, sem_t1, sem_e1):
    wid = lax.axis_index("s") * NUM_CORES + lax.axis_index("c")
    ai = lax.iota(jnp.int32, 16)

    tabs = (tab0, tab1)
    edgs = (edg0, edg1)
    sems = ((sem_t0, sem_e0), (sem_t1, sem_e1))

    def stage(s, slot):
        b = wid * spw + s
        ct = pltpu.async_copy(table_hbm.at[b], tabs[slot], sems[slot][0])
        ce = pltpu.async_copy(edges_hbm.at[b], edgs[slot], sems[slot][1])
        return (ct, ce)

    cps = stage(0, 0)
    for s in range(spw):
        slot = s & 1
        tab, edg = tabs[slot], edgs[slot]
        for c in cps:
            c.wait()
        if s + 1 < spw:
            cps = stage(s + 1, slot ^ 1)

        def per_chunk(ch, _):
            e_base = ch * CHUNK_IDX + ai * D
            cols = []
            for d in range(D):
                e_d = plsc.load_gather(edg, [e_base + d])
                cols.append(jnp.where(e_d < 0, A + ai, e_d))
            a0 = ch * CHUNK_ATOMS

            def per_f2(j, _):
                for f in (2 * j, 2 * j + 1):
                    base = f * AP
                    v = tab[pl.ds(base + a0, 16)]
                    for d in range(D):
                        v = v + plsc.load_gather(tab, [cols[d] + base])
                    out_v[pl.ds(f * A + a0, 16)] = v
                return 0

            lax.fori_loop(0, F // 2, per_f2, 0)
            return 0

        lax.fori_loop(0, CHUNKS_PER_SAMPLE, per_chunk, 0)
        b = wid * spw + s
        pltpu.sync_copy(out_v, out_hbm.at[b])


@functools.cache
def _sc_gather(nb):
    spw = nb // NUM_WORKERS
    return functools.partial(
        pl.kernel,
        mesh=plsc.VectorSubcoreMesh(core_axis_name="c", subcore_axis_name="s"),
        compiler_params=pltpu.CompilerParams(
            use_tc_tiling_on_sc=False, needs_layout_passes=False),
        out_type=jax.ShapeDtypeStruct((nb, F * A), jnp.float32),
        scratch_types=[
            pltpu.VMEM((F * AP,), jnp.float32),
            pltpu.VMEM((F * AP,), jnp.float32),
            pltpu.VMEM((A * D,), jnp.int32),
            pltpu.VMEM((A * D,), jnp.int32),
            pltpu.VMEM((F * A,), jnp.float32),
            pltpu.SemaphoreType.DMA,
            pltpu.SemaphoreType.DMA,
            pltpu.SemaphoreType.DMA,
            pltpu.SemaphoreType.DMA,
        ],
    )(functools.partial(_gather_body, spw))


ROWS_BLK = A


def _dense_body(neigh_ref, bonds_ref, edges_ref, wa_ref, wb_ref,
                bias_ref, out_ref):
    z = lax.dot_general(neigh_ref[...], wa_ref[...],
                        (((0,), (0,)), ((), ())),
                        preferred_element_type=jnp.float32)
    z = z + jnp.dot(bonds_ref[...], wb_ref[...],
                    preferred_element_type=jnp.float32)
    z = z + bias_ref[...]
    deg = jnp.sum((edges_ref[...] >= 0).astype(jnp.int32), axis=1, keepdims=True)
    acc = jnp.zeros((ROWS_BLK, C), jnp.float32)
    for d in range(D):
        acc = acc + jnp.where(deg == d, z[:, d * C:(d + 1) * C], 0.0)
    out_ref[...] = jnp.maximum(acc, 0.0)


NSPLIT = 1


def _dense_call(neigh_t, bonds_f, edges_f, wa, wb, bias, nrows):
    grid = (nrows // ROWS_BLK,)
    return pl.pallas_call(
        _dense_body,
        grid=grid,
        in_specs=[
            pl.BlockSpec((F, ROWS_BLK), lambda i: (i, 0)),
            pl.BlockSpec((ROWS_BLK, D * FB), lambda i: (i, 0)),
            pl.BlockSpec((ROWS_BLK, D), lambda i: (i, 0)),
            pl.BlockSpec((F, D * C), lambda i: (0, 0)),
            pl.BlockSpec((D * FB, D * C), lambda i: (0, 0)),
            pl.BlockSpec((1, D * C), lambda i: (0, 0)),
        ],
        out_specs=pl.BlockSpec((ROWS_BLK, C), lambda i: (i, 0)),
        out_shape=jax.ShapeDtypeStruct((nrows, C), jnp.float32),
    )(neigh_t, bonds_f, edges_f, wa, wb, bias)


def kernel(atoms, bonds, edges, W, b):
    atoms_tp = jnp.pad(atoms.transpose(0, 2, 1), ((0, 0), (0, 0), (0, AP - A)))
    atoms_tp = atoms_tp.reshape(B, F * AP)
    edges_sc = edges.reshape(B, A * D)

    bonds_f = bonds.reshape(B * A, D * FB)
    edges_f = edges.reshape(B * A, D)
    wa = jnp.transpose(W[:, :F, :], (1, 0, 2)).reshape(F, D * C)
    wb = jnp.tile(jnp.transpose(W[:, F:, :], (1, 0, 2)).reshape(FB, D * C), (D, 1))
    bias = b.reshape(1, D * C)

    nb = B // NSPLIT
    outs = []
    prev = None
    for k in range(NSPLIT):
        bs, rs = k * nb, k * nb * A
        a_k, e_k = atoms_tp[bs:bs + nb], edges_sc[bs:bs + nb]
        if prev is not None:
            a_k, e_k, _ = lax.optimization_barrier((a_k, e_k, prev))
        neigh_t = _sc_gather(nb)(a_k, e_k)
        prev = neigh_t
        neigh_t = neigh_t.reshape(nb * F, A)
        outs.append(_dense_call(neigh_t, bonds_f[rs:rs + nb * A],
                                edges_f[rs:rs + nb * A], wa, wb, bias, nb * A))
    out = jnp.concatenate(outs, axis=0) if NSPLIT > 1 else outs[0]
    return out.reshape(B, A, C)


# speedup vs baseline: 37.2693x; 1.0126x over previous
import functools

import jax
import jax.numpy as jnp
from jax import lax
from jax.experimental import pallas as pl
from jax.experimental.pallas import tpu as pltpu
from jax.experimental.pallas import tpu_sc as plsc

B, A, D, F, FB, C = 128, 512, 8, 64, 16, 64

NUM_CORES = 2
NUM_SUBCORES = 16
NUM_WORKERS = NUM_CORES * NUM_SUBCORES
SAMPLES_PER_WORKER = B // NUM_WORKERS
CHUNK_ATOMS = 16
CHUNK_IDX = CHUNK_ATOMS * D
CHUNKS_PER_SAMPLE = A // CHUNK_ATOMS
AP = A + 16


def _gather_body(spw, table_hbm, edges_hbm, out_hbm,
                 tab0, tab1, edg0, edg1, out_v,
                 sem_t0, sem_e0, sem_t1, sem_e1):
    wid = lax.axis_index("s") * NUM_CORES + lax.axis_index("c")
    ai = lax.iota(jnp.int32, 16)

    tabs = (tab0, tab1)
    edgs = (edg0, edg1)
    sems = ((sem_t0, sem_e0), (sem_t1, sem_e1))

    def stage(s, slot):
        b = wid * spw + s
        ct = pltpu.async_copy(table_hbm.at[b], tabs[slot], sems[slot][0])
        ce = pltpu.async_copy(edges_hbm.at[b], edgs[slot], sems[slot][1])
        return (ct, ce)

    cps = stage(0, 0)
    for s in range(spw):
        slot = s & 1
        tab, edg = tabs[slot], edgs[slot]
        for c in cps:
            c.wait()
        if s + 1 < spw:
            cps = stage(s + 1, slot ^ 1)

        def per_chunk(ch, _):
            e_base = ch * CHUNK_IDX + ai * D
            cols = []
            for d in range(D):
                e_d = plsc.load_gather(edg, [e_base + d])
                cols.append(jnp.where(e_d < 0, A + ai, e_d))
            a0 = ch * CHUNK_ATOMS

            def per_f2(j, _):
                for f in (2 * j, 2 * j + 1):
                    base = f * AP
                    g = [tab[pl.ds(base + a0, 16)]]
                    g += [plsc.load_gather(tab, [cols[d] + base])
                          for d in range(D)]
                    while len(g) > 1:
                        g = [g[i] + g[i + 1] for i in range(0, len(g) - 1, 2)] \
                            + ([g[-1]] if len(g) & 1 else [])
                    out_v[pl.ds(f * A + a0, 16)] = g[0]
                return 0

            lax.fori_loop(0, F // 2, per_f2, 0)
            return 0

        lax.fori_loop(0, CHUNKS_PER_SAMPLE, per_chunk, 0)
        b = wid * spw + s
        pltpu.sync_copy(out_v, out_hbm.at[b])


@functools.cache
def _sc_gather(nb):
    spw = nb // NUM_WORKERS
    return functools.partial(
        pl.kernel,
        mesh=plsc.VectorSubcoreMesh(core_axis_name="c", subcore_axis_name="s"),
        compiler_params=pltpu.CompilerParams(
            use_tc_tiling_on_sc=False, needs_layout_passes=False),
        out_type=jax.ShapeDtypeStruct((nb, F * A), jnp.float32),
        scratch_types=[
            pltpu.VMEM((F * AP,), jnp.float32),
            pltpu.VMEM((F * AP,), jnp.float32),
            pltpu.VMEM((A * D,), jnp.int32),
            pltpu.VMEM((A * D,), jnp.int32),
            pltpu.VMEM((F * A,), jnp.float32),
            pltpu.SemaphoreType.DMA,
            pltpu.SemaphoreType.DMA,
            pltpu.SemaphoreType.DMA,
            pltpu.SemaphoreType.DMA,
        ],
    )(functools.partial(_gather_body, spw))


ROWS_BLK = A


def _dense_body(neigh_ref, bonds_ref, edges_ref, wa_ref, wb_ref,
                bias_ref, out_ref):
    z = lax.dot_general(neigh_ref[...], wa_ref[...],
                        (((0,), (0,)), ((), ())),
                        preferred_element_type=jnp.float32)
    z = z + jnp.dot(bonds_ref[...], wb_ref[...],
                    preferred_element_type=jnp.float32)
    z = z + bias_ref[...]
    deg = jnp.sum((edges_ref[...] >= 0).astype(jnp.int32), axis=1, keepdims=True)
    acc = jnp.zeros((ROWS_BLK, C), jnp.float32)
    for d in range(D):
        acc = acc + jnp.where(deg == d, z[:, d * C:(d + 1) * C], 0.0)
    out_ref[...] = jnp.maximum(acc, 0.0)


NSPLIT = 1


def _dense_call(neigh_t, bonds_f, edges_f, wa, wb, bias, nrows):
    grid = (nrows // ROWS_BLK,)
    return pl.pallas_call(
        _dense_body,
        grid=grid,
        in_specs=[
            pl.BlockSpec((F, ROWS_BLK), lambda i: (i, 0)),
            pl.BlockSpec((ROWS_BLK, D * FB), lambda i: (i, 0)),
            pl.BlockSpec((ROWS_BLK, D), lambda i: (i, 0)),
            pl.BlockSpec((F, D * C), lambda i: (0, 0)),
            pl.BlockSpec((D * FB, D * C), lambda i: (0, 0)),
            pl.BlockSpec((1, D * C), lambda i: (0, 0)),
        ],
        out_specs=pl.BlockSpec((ROWS_BLK, C), lambda i: (i, 0)),
        out_shape=jax.ShapeDtypeStruct((nrows, C), jnp.float32),
    )(neigh_t, bonds_f, edges_f, wa, wb, bias)


def kernel(atoms, bonds, edges, W, b):
    atoms_tp = jnp.pad(atoms.transpose(0, 2, 1), ((0, 0), (0, 0), (0, AP - A)))
    atoms_tp = atoms_tp.reshape(B, F * AP)
    edges_sc = edges.reshape(B, A * D)

    bonds_f = bonds.reshape(B * A, D * FB)
    edges_f = edges.reshape(B * A, D)
    wa = jnp.transpose(W[:, :F, :], (1, 0, 2)).reshape(F, D * C)
    wb = jnp.tile(jnp.transpose(W[:, F:, :], (1, 0, 2)).reshape(FB, D * C), (D, 1))
    bias = b.reshape(1, D * C)

    nb = B // NSPLIT
    outs = []
    prev = None
    for k in range(NSPLIT):
        bs, rs = k * nb, k * nb * A
        a_k, e_k = atoms_tp[bs:bs + nb], edges_sc[bs:bs + nb]
        if prev is not None:
            a_k, e_k, _ = lax.optimization_barrier((a_k, e_k, prev))
        neigh_t = _sc_gather(nb)(a_k, e_k)
        prev = neigh_t
        neigh_t = neigh_t.reshape(nb * F, A)
        outs.append(_dense_call(neigh_t, bonds_f[rs:rs + nb * A],
                                edges_f[rs:rs + nb * A], wa, wb, bias, nb * A))
    out = jnp.concatenate(outs, axis=0) if NSPLIT > 1 else outs[0]
    return out.reshape(B, A, C)


# speedup vs baseline: 38.5347x; 1.0340x over previous
import functools

import jax
import jax.numpy as jnp
from jax import lax
from jax.experimental import pallas as pl
from jax.experimental.pallas import tpu as pltpu
from jax.experimental.pallas import tpu_sc as plsc

B, A, D, F, FB, C = 128, 512, 8, 64, 16, 64

NUM_CORES = 2
NUM_SUBCORES = 16
NUM_WORKERS = NUM_CORES * NUM_SUBCORES


def _tree_sum(g):
    while len(g) > 1:
        g = [g[i] + g[i + 1] for i in range(0, len(g) - 1, 2)] \
            + ([g[-1]] if len(g) & 1 else [])
    return g[0]


def _gather_body(spw, table_hbm, edges_hbm, out_hbm,
                 tab0, tab1, edg0, edg1, out_v,
                 sem_t0, sem_e0, sem_t1, sem_e1):
    wid = lax.axis_index("s") * NUM_CORES + lax.axis_index("c")

    tabs = (tab0, tab1)
    edgs = (edg0, edg1)
    sems = ((sem_t0, sem_e0), (sem_t1, sem_e1))

    for t in tabs:
        for j in range(F // 16):
            t[pl.ds(A * F + j * 16, 16)] = jnp.zeros((16,), jnp.float32)

    def stage(s, slot):
        b = wid * spw + s
        ct = pltpu.async_copy(table_hbm.at[b], tabs[slot].at[pl.ds(0, A * F)],
                              sems[slot][0])
        ce = pltpu.async_copy(edges_hbm.at[b], edgs[slot], sems[slot][1])
        return (ct, ce)

    cps = stage(0, 0)
    for s in range(spw):
        slot = s & 1
        tab, edg = tabs[slot], edgs[slot]
        for c in cps:
            c.wait()
        if s + 1 < spw:
            cps = stage(s + 1, slot ^ 1)

        def per_pair(p, _):
            ev = edg[pl.ds(p * 16, 16)]
            for k in range(2):
                base0 = (p * 2 + k) * F
                bases = []
                for d in range(D):
                    e = ev[k * D + d]
                    bases.append(jnp.where(e < 0, A * F, e * F))
                for j in range(F // 16):
                    off = j * 16
                    g = [tab[pl.ds(base0 + off, 16)]]
                    g += [tab[pl.ds(bases[d] + off, 16)] for d in range(D)]
                    out_v[pl.ds(base0 + off, 16)] = _tree_sum(g)
            return 0

        lax.fori_loop(0, A // 2, per_pair, 0)
        b = wid * spw + s
        pltpu.sync_copy(out_v, out_hbm.at[b])


@functools.cache
def _sc_gather(nb):
    spw = nb // NUM_WORKERS
    return functools.partial(
        pl.kernel,
        mesh=plsc.VectorSubcoreMesh(core_axis_name="c", subcore_axis_name="s"),
        compiler_params=pltpu.CompilerParams(
            use_tc_tiling_on_sc=False, needs_layout_passes=False),
        out_type=jax.ShapeDtypeStruct((nb, A * F), jnp.float32),
        scratch_types=[
            pltpu.VMEM(((A + 1) * F,), jnp.float32),
            pltpu.VMEM(((A + 1) * F,), jnp.float32),
            pltpu.VMEM((A * D,), jnp.int32),
            pltpu.VMEM((A * D,), jnp.int32),
            pltpu.VMEM((A * F,), jnp.float32),
            pltpu.SemaphoreType.DMA,
            pltpu.SemaphoreType.DMA,
            pltpu.SemaphoreType.DMA,
            pltpu.SemaphoreType.DMA,
        ],
    )(functools.partial(_gather_body, spw))


ROWS_BLK = 512


def _dense_body(neigh_ref, bonds_ref, edges_ref, wa_ref, wb_ref,
                bias_ref, out_ref):
    z = jnp.dot(neigh_ref[...], wa_ref[...], preferred_element_type=jnp.float32)
    z = z + jnp.dot(bonds_ref[...], wb_ref[...],
                    preferred_element_type=jnp.float32)
    z = z + bias_ref[...]
    deg = jnp.sum((edges_ref[...] >= 0).astype(jnp.int32), axis=1, keepdims=True)
    acc = jnp.zeros((ROWS_BLK, C), jnp.float32)
    for d in range(D):
        acc = acc + jnp.where(deg == d, z[:, d * C:(d + 1) * C], 0.0)
    out_ref[...] = jnp.maximum(acc, 0.0)


def kernel(atoms, bonds, edges, W, b):
    atoms_sc = atoms.reshape(B, A * F)
    edges_sc = edges.reshape(B, A * D)

    neigh = _sc_gather(B)(atoms_sc, edges_sc).reshape(B * A, F)

    bonds_f = bonds.reshape(B * A, D * FB)
    edges_f = edges.reshape(B * A, D)
    wa = jnp.transpose(W[:, :F, :], (1, 0, 2)).reshape(F, D * C)
    wb = jnp.tile(jnp.transpose(W[:, F:, :], (1, 0, 2)).reshape(FB, D * C), (D, 1))
    bias = b.reshape(1, D * C)

    grid = (B * A // ROWS_BLK,)
    out = pl.pallas_call(
        _dense_body,
        grid=grid,
        in_specs=[
            pl.BlockSpec((ROWS_BLK, F), lambda i: (i, 0)),
            pl.BlockSpec((ROWS_BLK, D * FB), lambda i: (i, 0)),
            pl.BlockSpec((ROWS_BLK, D), lambda i: (i, 0)),
            pl.BlockSpec((F, D * C), lambda i: (0, 0)),
            pl.BlockSpec((D * FB, D * C), lambda i: (0, 0)),
            pl.BlockSpec((1, D * C), lambda i: (0, 0)),
        ],
        out_specs=pl.BlockSpec((ROWS_BLK, C), lambda i: (i, 0)),
        out_shape=jax.ShapeDtypeStruct((B * A, C), jnp.float32),
    )(neigh, bonds_f, edges_f, wa, wb, bias)

    return out.reshape(B, A, C)


# speedup vs baseline: 41.5420x; 1.0780x over previous
import functools

import jax
import jax.numpy as jnp
from jax import lax
from jax.experimental import pallas as pl
from jax.experimental.pallas import tpu as pltpu
from jax.experimental.pallas import tpu_sc as plsc

B, A, D, F, FB, C = 128, 512, 8, 64, 16, 64

NUM_CORES = 2
NUM_SUBCORES = 16
NUM_WORKERS = NUM_CORES * NUM_SUBCORES


def _tree_sum(g):
    while len(g) > 1:
        g = [g[i] + g[i + 1] for i in range(0, len(g) - 1, 2)] \
            + ([g[-1]] if len(g) & 1 else [])
    return g[0]


def _gather_body(spw, table_hbm, edges_hbm, out_hbm,
                 tab0, tab1, edg0, edg1, out_v,
                 sem_t0, sem_e0, sem_t1, sem_e1):
    wid = lax.axis_index("s") * NUM_CORES + lax.axis_index("c")

    tabs = (tab0, tab1)
    edgs = (edg0, edg1)
    sems = ((sem_t0, sem_e0), (sem_t1, sem_e1))

    for t in tabs:
        for j in range(F // 16):
            t[A, pl.ds(j * 16, 16)] = jnp.zeros((16,), jnp.float32)

    def stage(s, slot):
        b = wid * spw + s
        ct = pltpu.async_copy(table_hbm.at[b], tabs[slot].at[pl.ds(0, A)],
                              sems[slot][0])
        ce = pltpu.async_copy(edges_hbm.at[b], edgs[slot], sems[slot][1])
        return (ct, ce)

    cps = stage(0, 0)
    for s in range(spw):
        slot = s & 1
        tab, edg = tabs[slot], edgs[slot]
        for c in cps:
            c.wait()
        if s + 1 < spw:
            cps = stage(s + 1, slot ^ 1)

        def per_pair(p, _):
            ev = edg[pl.ds(p * 16, 16)]
            for k in range(2):
                a = p * 2 + k
                rows = []
                for d in range(D):
                    e = ev[k * D + d]
                    rows.append(jnp.where(e < 0, A, e))
                for j in range(F // 16):
                    off = pl.ds(j * 16, 16)
                    g = [tab[a, off]]
                    g += [tab[rows[d], off] for d in range(D)]
                    out_v[a, off] = _tree_sum(g)
            return 0

        lax.fori_loop(0, A // 2, per_pair, 0)
        b = wid * spw + s
        pltpu.sync_copy(out_v, out_hbm.at[b])


@functools.cache
def _sc_gather(nb):
    spw = nb // NUM_WORKERS
    return functools.partial(
        pl.kernel,
        mesh=plsc.VectorSubcoreMesh(core_axis_name="c", subcore_axis_name="s"),
        compiler_params=pltpu.CompilerParams(
            use_tc_tiling_on_sc=False, needs_layout_passes=False),
        out_type=jax.ShapeDtypeStruct((nb, A, F), jnp.float32),
        scratch_types=[
            pltpu.VMEM((A + 1, F), jnp.float32),
            pltpu.VMEM((A + 1, F), jnp.float32),
            pltpu.VMEM((A * D,), jnp.int32),
            pltpu.VMEM((A * D,), jnp.int32),
            pltpu.VMEM((A, F), jnp.float32),
            pltpu.SemaphoreType.DMA,
            pltpu.SemaphoreType.DMA,
            pltpu.SemaphoreType.DMA,
            pltpu.SemaphoreType.DMA,
        ],
    )(functools.partial(_gather_body, spw))


ROWS_BLK = 512


def _dense_body(neigh_ref, bonds_ref, edges_ref, wa_ref, wb_ref,
                bias_ref, out_ref):
    bf = jnp.bfloat16
    z = jnp.dot(neigh_ref[...].astype(bf), wa_ref[...].astype(bf),
                preferred_element_type=jnp.float32)
    z = z + jnp.dot(bonds_ref[...].astype(bf), wb_ref[...].astype(bf),
                    preferred_element_type=jnp.float32)
    z = z + bias_ref[...]
    deg = jnp.sum((edges_ref[...] >= 0).astype(jnp.int32), axis=1, keepdims=True)
    acc = jnp.zeros((ROWS_BLK, C), jnp.float32)
    for d in range(D):
        acc = acc + jnp.where(deg == d, z[:, d * C:(d + 1) * C], 0.0)
    out_ref[...] = jnp.maximum(acc, 0.0)


def kernel(atoms, bonds, edges, W, b):
    edges_sc = edges.reshape(B, A * D)

    neigh = _sc_gather(B)(atoms, edges_sc).reshape(B * A, F)

    bonds_f = bonds.reshape(B * A, D * FB)
    edges_f = edges.reshape(B * A, D)
    wa = jnp.transpose(W[:, :F, :], (1, 0, 2)).reshape(F, D * C)
    wb = jnp.tile(jnp.transpose(W[:, F:, :], (1, 0, 2)).reshape(FB, D * C), (D, 1))
    bias = b.reshape(1, D * C)

    grid = (B * A // ROWS_BLK,)
    out = pl.pallas_call(
        _dense_body,
        grid=grid,
        in_specs=[
            pl.BlockSpec((ROWS_BLK, F), lambda i: (i, 0)),
            pl.BlockSpec((ROWS_BLK, D * FB), lambda i: (i, 0)),
            pl.BlockSpec((ROWS_BLK, D), lambda i: (i, 0)),
            pl.BlockSpec((F, D * C), lambda i: (0, 0)),
            pl.BlockSpec((D * FB, D * C), lambda i: (0, 0)),
            pl.BlockSpec((1, D * C), lambda i: (0, 0)),
        ],
        out_specs=pl.BlockSpec((ROWS_BLK, C), lambda i: (i, 0)),
        out_shape=jax.ShapeDtypeStruct((B * A, C), jnp.float32),
    )(neigh, bonds_f, edges_f, wa, wb, bias)

    return out.reshape(B, A, C)


# speedup vs baseline: 45.8744x; 1.1043x over previous
import functools

import jax
import jax.numpy as jnp
from jax import lax
from jax.experimental import pallas as pl
from jax.experimental.pallas import tpu as pltpu
from jax.experimental.pallas import tpu_sc as plsc

B, A, D, F, FB, C = 128, 512, 8, 64, 16, 64

NUM_CORES = 2
NUM_SUBCORES = 16
NUM_WORKERS = NUM_CORES * NUM_SUBCORES


def _tree_sum(g):
    while len(g) > 1:
        g = [g[i] + g[i + 1] for i in range(0, len(g) - 1, 2)] \
            + ([g[-1]] if len(g) & 1 else [])
    return g[0]


def _gather_body(spw, table_hbm, edges_hbm, out_hbm,
                 tab0, tab1, edg0, edg1, out_v,
                 sem_t0, sem_e0, sem_t1, sem_e1):
    wid = lax.axis_index("s") * NUM_CORES + lax.axis_index("c")

    tabs = (tab0, tab1)
    edgs = (edg0, edg1)
    sems = ((sem_t0, sem_e0), (sem_t1, sem_e1))

    for t in tabs:
        for j in range(F // 16):
            t[A, pl.ds(j * 16, 16)] = jnp.zeros((16,), jnp.float32)

    def stage(s, slot):
        b = wid * spw + s
        ct = pltpu.async_copy(table_hbm.at[b], tabs[slot].at[pl.ds(0, A)],
                              sems[slot][0])
        ce = pltpu.async_copy(edges_hbm.at[b], edgs[slot], sems[slot][1])
        return (ct, ce)

    cps = stage(0, 0)
    for s in range(spw):
        slot = s & 1
        tab, edg = tabs[slot], edgs[slot]
        for c in cps:
            c.wait()
        if s + 1 < spw:
            cps = stage(s + 1, slot ^ 1)

        def per_pair(p, _):
            ev = edg[pl.ds(p * 16, 16)]
            for k in range(2):
                a = p * 2 + k
                rows = []
                for d in range(D):
                    e = ev[k * D + d]
                    rows.append(jnp.where(e < 0, A, e))
                for j in range(F // 16):
                    off = pl.ds(j * 16, 16)
                    g = [tab[a, off]]
                    g += [tab[rows[d], off] for d in range(D)]
                    out_v[a, off] = _tree_sum(g)
            return 0

        lax.fori_loop(0, A // 2, per_pair, 0)
        b = wid * spw + s
        pltpu.sync_copy(out_v, out_hbm.at[b])


@functools.cache
def _sc_gather(nb):
    spw = nb // NUM_WORKERS
    return functools.partial(
        pl.kernel,
        mesh=plsc.VectorSubcoreMesh(core_axis_name="c", subcore_axis_name="s"),
        compiler_params=pltpu.CompilerParams(
            use_tc_tiling_on_sc=False, needs_layout_passes=False),
        out_type=jax.ShapeDtypeStruct((nb, A, F), jnp.float32),
        scratch_types=[
            pltpu.VMEM((A + 1, F), jnp.float32),
            pltpu.VMEM((A + 1, F), jnp.float32),
            pltpu.VMEM((A * D,), jnp.int32),
            pltpu.VMEM((A * D,), jnp.int32),
            pltpu.VMEM((A, F), jnp.float32),
            pltpu.SemaphoreType.DMA,
            pltpu.SemaphoreType.DMA,
            pltpu.SemaphoreType.DMA,
            pltpu.SemaphoreType.DMA,
        ],
    )(functools.partial(_gather_body, spw))


ROWS_BLK = 1024


def _dense_body(neigh_ref, bonds_ref, edges_ref, wa_ref, wb_ref,
                bias_ref, out_ref):
    z = jnp.dot(neigh_ref[...], wa_ref[...], preferred_element_type=jnp.float32)
    z = z + jnp.dot(bonds_ref[...], wb_ref[...],
                    preferred_element_type=jnp.float32)
    z = z + bias_ref[...]
    deg = jnp.sum((edges_ref[...] >= 0).astype(jnp.int32), axis=1, keepdims=True)
    acc = jnp.zeros((ROWS_BLK, C), jnp.float32)
    for d in range(D):
        acc = acc + jnp.where(deg == d, z[:, d * C:(d + 1) * C], 0.0)
    out_ref[...] = jnp.maximum(acc, 0.0)


def kernel(atoms, bonds, edges, W, b):
    edges_sc = edges.reshape(B, A * D)

    neigh = _sc_gather(B)(atoms, edges_sc).reshape(B * A, F)

    bonds_f = bonds.reshape(B * A, D * FB)
    edges_f = edges.reshape(B * A, D)
    wa = jnp.transpose(W[:, :F, :], (1, 0, 2)).reshape(F, D * C)
    wb = jnp.tile(jnp.transpose(W[:, F:, :], (1, 0, 2)).reshape(FB, D * C), (D, 1))
    bias = b.reshape(1, D * C)

    grid = (B * A // ROWS_BLK,)
    out = pl.pallas_call(
        _dense_body,
        grid=grid,
        in_specs=[
            pl.BlockSpec((ROWS_BLK, F), lambda i: (i, 0)),
            pl.BlockSpec((ROWS_BLK, D * FB), lambda i: (i, 0)),
            pl.BlockSpec((ROWS_BLK, D), lambda i: (i, 0)),
            pl.BlockSpec((F, D * C), lambda i: (0, 0)),
            pl.BlockSpec((D * FB, D * C), lambda i: (0, 0)),
            pl.BlockSpec((1, D * C), lambda i: (0, 0)),
        ],
        out_specs=pl.BlockSpec((ROWS_BLK, C), lambda i: (i, 0)),
        out_shape=jax.ShapeDtypeStruct((B * A, C), jnp.float32),
    )(neigh, bonds_f, edges_f, wa, wb, bias)

    return out.reshape(B, A, C)


# speedup vs baseline: 48.2077x; 1.0509x over previous
import functools

import jax
import jax.numpy as jnp
from jax import lax
from jax.experimental import pallas as pl
from jax.experimental.pallas import tpu as pltpu
from jax.experimental.pallas import tpu_sc as plsc

B, A, D, F, FB, C = 128, 512, 8, 64, 16, 64

NUM_CORES = 2
NUM_SUBCORES = 16
NUM_WORKERS = NUM_CORES * NUM_SUBCORES


def _tree_sum(g):
    while len(g) > 1:
        g = [g[i] + g[i + 1] for i in range(0, len(g) - 1, 2)] \
            + ([g[-1]] if len(g) & 1 else [])
    return g[0]


def _gather_body(spw, table_hbm, edges_hbm, out_hbm,
                 tab0, tab1, edg0, edg1, out_v,
                 sem_t0, sem_e0, sem_t1, sem_e1):
    wid = lax.axis_index("s") * NUM_CORES + lax.axis_index("c")

    tabs = (tab0, tab1)
    edgs = (edg0, edg1)
    sems = ((sem_t0, sem_e0), (sem_t1, sem_e1))

    for t in tabs:
        for j in range(F // 16):
            t[A, pl.ds(j * 16, 16)] = jnp.zeros((16,), jnp.float32)

    def stage(s, slot):
        b = wid * spw + s
        ct = pltpu.async_copy(table_hbm.at[b], tabs[slot].at[pl.ds(0, A)],
                              sems[slot][0])
        ce = pltpu.async_copy(edges_hbm.at[b], edgs[slot], sems[slot][1])
        return (ct, ce)

    cps = stage(0, 0)
    for s in range(spw):
        slot = s & 1
        tab, edg = tabs[slot], edgs[slot]
        for c in cps:
            c.wait()
        if s + 1 < spw:
            cps = stage(s + 1, slot ^ 1)

        def per_pair(p, _):
            ev = edg[pl.ds(p * 16, 16)]
            for k in range(2):
                a = p * 2 + k
                rows = []
                for d in range(D):
                    e = ev[k * D + d]
                    rows.append(jnp.where(e < 0, A, e))
                for j in range(F // 16):
                    off = pl.ds(j * 16, 16)
                    g = [tab[a, off]]
                    g += [tab[rows[d], off] for d in range(D)]
                    out_v[a, off] = _tree_sum(g)
            return 0

        lax.fori_loop(0, A // 2, per_pair, 0)
        b = wid * spw + s
        pltpu.sync_copy(out_v, out_hbm.at[b])


@functools.cache
def _sc_gather(nb):
    spw = nb // NUM_WORKERS
    return functools.partial(
        pl.kernel,
        mesh=plsc.VectorSubcoreMesh(core_axis_name="c", subcore_axis_name="s"),
        compiler_params=pltpu.CompilerParams(
            use_tc_tiling_on_sc=False, needs_layout_passes=False),
        out_type=jax.ShapeDtypeStruct((nb, A, F), jnp.float32),
        scratch_types=[
            pltpu.VMEM((A + 1, F), jnp.float32),
            pltpu.VMEM((A + 1, F), jnp.float32),
            pltpu.VMEM((A * D,), jnp.int32),
            pltpu.VMEM((A * D,), jnp.int32),
            pltpu.VMEM((A, F), jnp.float32),
            pltpu.SemaphoreType.DMA,
            pltpu.SemaphoreType.DMA,
            pltpu.SemaphoreType.DMA,
            pltpu.SemaphoreType.DMA,
        ],
    )(functools.partial(_gather_body, spw))


ROWS_BLK = 2048


def _dense_body(neigh_ref, bonds_ref, edges_ref, wa_ref, wb_ref,
                bias_ref, out_ref):
    z = jnp.dot(neigh_ref[...], wa_ref[...], preferred_element_type=jnp.float32)
    z = z + jnp.dot(bonds_ref[...], wb_ref[...],
                    preferred_element_type=jnp.float32)
    z = z + bias_ref[...]
    deg = jnp.sum((edges_ref[...] >= 0).astype(jnp.int32), axis=1, keepdims=True)
    acc = jnp.zeros((ROWS_BLK, C), jnp.float32)
    for d in range(D):
        acc = acc + jnp.where(deg == d, z[:, d * C:(d + 1) * C], 0.0)
    out_ref[...] = jnp.maximum(acc, 0.0)


def kernel(atoms, bonds, edges, W, b):
    edges_sc = edges.reshape(B, A * D)

    neigh = _sc_gather(B)(atoms, edges_sc).reshape(B * A, F)

    bonds_f = bonds.reshape(B * A, D * FB)
    edges_f = edges.reshape(B * A, D)
    wa = jnp.transpose(W[:, :F, :], (1, 0, 2)).reshape(F, D * C)
    wb = jnp.tile(jnp.transpose(W[:, F:, :], (1, 0, 2)).reshape(FB, D * C), (D, 1))
    bias = b.reshape(1, D * C)

    grid = (B * A // ROWS_BLK,)
    out = pl.pallas_call(
        _dense_body,
        grid=grid,
        in_specs=[
            pl.BlockSpec((ROWS_BLK, F), lambda i: (i, 0)),
            pl.BlockSpec((ROWS_BLK, D * FB), lambda i: (i, 0)),
            pl.BlockSpec((ROWS_BLK, D), lambda i: (i, 0)),
            pl.BlockSpec((F, D * C), lambda i: (0, 0)),
            pl.BlockSpec((D * FB, D * C), lambda i: (0, 0)),
            pl.BlockSpec((1, D * C), lambda i: (0, 0)),
        ],
        out_specs=pl.BlockSpec((ROWS_BLK, C), lambda i: (i, 0)),
        out_shape=jax.ShapeDtypeStruct((B * A, C), jnp.float32),
    )(neigh, bonds_f, edges_f, wa, wb, bias)

    return out.reshape(B, A, C)


# speedup vs baseline: 48.8173x; 1.0126x over previous
import functools

import jax
import jax.numpy as jnp
from jax import lax
from jax.experimental import pallas as pl
from jax.experimental.pallas import tpu as pltpu
from jax.experimental.pallas import tpu_sc as plsc

B, A, D, F, FB, C = 128, 512, 8, 64, 16, 64

NUM_CORES = 2
NUM_SUBCORES = 16
NUM_WORKERS = NUM_CORES * NUM_SUBCORES


def _tree_sum(g):
    while len(g) > 1:
        g = [g[i] + g[i + 1] for i in range(0, len(g) - 1, 2)] \
            + ([g[-1]] if len(g) & 1 else [])
    return g[0]


def _gather_body(spw, table_hbm, edges_hbm, out_hbm,
                 tab0, tab1, edg0, edg1, out_v,
                 sem_t0, sem_e0, sem_t1, sem_e1):
    wid = lax.axis_index("s") * NUM_CORES + lax.axis_index("c")

    tabs = (tab0, tab1)
    edgs = (edg0, edg1)
    sems = ((sem_t0, sem_e0), (sem_t1, sem_e1))

    for t in tabs:
        for j in range(F // 16):
            t[A, pl.ds(j * 16, 16)] = jnp.zeros((16,), jnp.float32)

    def stage(s, slot):
        b = wid * spw + s
        ct = pltpu.async_copy(table_hbm.at[b], tabs[slot].at[pl.ds(0, A)],
                              sems[slot][0])
        ce = pltpu.async_copy(edges_hbm.at[b], edgs[slot], sems[slot][1])
        return (ct, ce)

    cps = stage(0, 0)
    for s in range(spw):
        slot = s & 1
        tab, edg = tabs[slot], edgs[slot]
        for c in cps:
            c.wait()
        if s + 1 < spw:
            cps = stage(s + 1, slot ^ 1)

        def per_pair(p, _):
            ev = edg[pl.ds(p * 16, 16)]
            for k in range(2):
                a = p * 2 + k
                rows = []
                for d in range(D):
                    e = ev[k * D + d]
                    rows.append(jnp.where(e < 0, A, e))
                for j in range(F // 16):
                    off = pl.ds(j * 16, 16)
                    g = [tab[a, off]]
                    g += [tab[rows[d], off] for d in range(D)]
                    out_v[a, off] = _tree_sum(g)
            return 0

        lax.fori_loop(0, A // 2, per_pair, 0)
        b = wid * spw + s
        pltpu.sync_copy(out_v, out_hbm.at[b])


@functools.cache
def _sc_gather(nb):
    spw = nb // NUM_WORKERS
    return functools.partial(
        pl.kernel,
        mesh=plsc.VectorSubcoreMesh(core_axis_name="c", subcore_axis_name="s"),
        compiler_params=pltpu.CompilerParams(
            use_tc_tiling_on_sc=False, needs_layout_passes=False),
        out_type=jax.ShapeDtypeStruct((nb, A, F), jnp.float32),
        scratch_types=[
            pltpu.VMEM((A + 1, F), jnp.float32),
            pltpu.VMEM((A + 1, F), jnp.float32),
            pltpu.VMEM((A * D,), jnp.int32),
            pltpu.VMEM((A * D,), jnp.int32),
            pltpu.VMEM((A, F), jnp.float32),
            pltpu.SemaphoreType.DMA,
            pltpu.SemaphoreType.DMA,
            pltpu.SemaphoreType.DMA,
            pltpu.SemaphoreType.DMA,
        ],
    )(functools.partial(_gather_body, spw))


ROWS_BLK = 4096


def _dense_body(neigh_ref, bonds_ref, edges_ref, wa_ref, wb_ref,
                bias_ref, out_ref):
    z = jnp.dot(neigh_ref[...], wa_ref[...], preferred_element_type=jnp.float32)
    z = z + jnp.dot(bonds_ref[...], wb_ref[...],
                    preferred_element_type=jnp.float32)
    z = z + bias_ref[...]
    deg = jnp.sum((edges_ref[...] >= 0).astype(jnp.int32), axis=1, keepdims=True)
    acc = jnp.zeros((ROWS_BLK, C), jnp.float32)
    for d in range(D):
        acc = acc + jnp.where(deg == d, z[:, d * C:(d + 1) * C], 0.0)
    out_ref[...] = jnp.maximum(acc, 0.0)


def kernel(atoms, bonds, edges, W, b):
    edges_sc = edges.reshape(B, A * D)

    neigh = _sc_gather(B)(atoms, edges_sc).reshape(B * A, F)

    bonds_f = bonds.reshape(B * A, D * FB)
    edges_f = edges.reshape(B * A, D)
    wa = jnp.transpose(W[:, :F, :], (1, 0, 2)).reshape(F, D * C)
    wb = jnp.tile(jnp.transpose(W[:, F:, :], (1, 0, 2)).reshape(FB, D * C), (D, 1))
    bias = b.reshape(1, D * C)

    grid = (B * A // ROWS_BLK,)
    out = pl.pallas_call(
        _dense_body,
        grid=grid,
        in_specs=[
            pl.BlockSpec((ROWS_BLK, F), lambda i: (i, 0)),
            pl.BlockSpec((ROWS_BLK, D * FB), lambda i: (i, 0)),
            pl.BlockSpec((ROWS_BLK, D), lambda i: (i, 0)),
            pl.BlockSpec((F, D * C), lambda i: (0, 0)),
            pl.BlockSpec((D * FB, D * C), lambda i: (0, 0)),
            pl.BlockSpec((1, D * C), lambda i: (0, 0)),
        ],
        out_specs=pl.BlockSpec((ROWS_BLK, C), lambda i: (i, 0)),
        out_shape=jax.ShapeDtypeStruct((B * A, C), jnp.float32),
    )(neigh, bonds_f, edges_f, wa, wb, bias)

    return out.reshape(B, A, C)
